# baseline (speedup 1.0000x reference)
import functools

import jax
import jax.numpy as jnp
import numpy as np
from jax import lax
from jax.experimental import pallas as pl
from jax.experimental.pallas import tpu as pltpu
from jax.experimental.pallas import tpu_sc as plsc

N = 10000
E = 320000
NC, NS, LANES = 2, 16, 16
NW = NC * NS
K = 80
NPAD = 10240
RPT = NPAD // NS
HD = 64
NBUF = 4


def _sc_agg(split_edges_across_cores):
    if split_edges_across_cores:
        nch = E // NW // K
        nh_sup = 1
    else:
        nch = E // NS // K
        nh_sup = 2
    mesh = plsc.VectorSubcoreMesh(core_axis_name="c", subcore_axis_name="s")

    @functools.partial(
        pl.kernel,
        out_type=jax.ShapeDtypeStruct((NC, NPAD, HD), jnp.float32),
        mesh=mesh,
        compiler_params=pltpu.CompilerParams(use_tc_tiling_on_sc=False,
                                             needs_layout_passes=False),
        scratch_types=[
            pltpu.VMEM((nch, K), jnp.int32),
            pltpu.VMEM((nch, K), jnp.int32),
            pltpu.VMEM((nch, K), jnp.float32),
            pltpu.VMEM((NBUF, K, HD), jnp.bfloat16),
            pltpu.VMEM((2, K, HD), jnp.float32),
            pltpu.VMEM_SHARED((NPAD, HD), jnp.float32),
            pltpu.SemaphoreType.DMA,
            [pltpu.SemaphoreType.DMA for _ in range(NBUF)],
            [pltpu.SemaphoreType.DMA for _ in range(NBUF)],
        ],
    )
    def agg(sup, srcr, tgtr, mr, out, srcv, tgtv, mv, rows, frows, acc,
            sem_i, gsems, ssems):
        cid = lax.axis_index("c")
        sid = lax.axis_index("s")
        r0 = sid * RPT
        if split_edges_across_cores:
            erow = sid * NC + cid
            suph = sup.at[0]
        else:
            erow = sid
            suph = sup.at[cid]

        cp_s = pltpu.async_copy(srcr.at[erow], srcv, sem_i)
        cp_t = pltpu.async_copy(tgtr.at[erow], tgtv, sem_i)
        cp_m = pltpu.async_copy(mr.at[erow], mv, sem_i)

        zv = jnp.zeros((LANES,), jnp.float32)

        @pl.loop(0, K)
        def _(e):
            for j in range(HD // LANES):
                frows[0, e, pl.ds(j * LANES, LANES)] = zv

        for j in range(RPT // K):
            pltpu.sync_copy(frows.at[0], acc.at[pl.ds(r0 + j * K, K)])

        cp_s.wait()
        cp_t.wait()
        cp_m.wait()
        plsc.subcore_barrier()

        def start_gather(c, b):
            pltpu.async_copy(suph.at[srcv.at[c]], rows.at[b], gsems[b])

        def wait_gather(c, b):
            pltpu.make_async_copy(suph.at[srcv.at[c]], rows.at[b],
                                  gsems[b]).wait()

        def start_scatter(c, bf):
            pltpu.async_copy(frows.at[bf], acc.at[tgtv.at[c]], ssems[bf],
                             add=True)

        def wait_scatter(c, bf):
            pltpu.make_async_copy(frows.at[bf], acc.at[tgtv.at[c]],
                                  ssems[bf]).wait()

        def scale(c, b, bf):
            @plsc.parallel_loop(0, K // LANES, unroll=2)
            def _(g):
                mvec = mv[c, pl.ds(g * LANES, LANES)]
                for i in range(LANES):
                    e = g * LANES + i
                    m = jnp.full((LANES,), mvec[i])
                    for j in range(HD // (2 * LANES)):
                        v = rows[b, e, pl.ds(j * 2 * LANES, 2 * LANES)]
                        lo, hi = plsc.unpack(
                            v, format=plsc.PackFormat.INTERLEAVED)
                        frows[bf, e, pl.ds(j * 2 * LANES, LANES)] = lo * m
                        frows[bf, e, pl.ds(j * 2 * LANES + LANES, LANES)] = (
                            hi * m)

        def step(c, b):
            bf = b % 2
            wait_gather(c, b)

            @pl.when(c >= 2)
            def _():
                wait_scatter(c - 2, bf)

            scale(c, b, bf)
            start_scatter(c, bf)

            @pl.when(c + 2 < nch)
            def _():
                start_gather(c + 2, (b + 2) % NBUF)

        start_gather(0, 0)
        start_gather(1, 1)

        @pl.loop(0, nch - (nch % NBUF), step=NBUF)
        def _(c):
            for db in range(NBUF):
                step(c + db, db)

        for c_tail in range(nch - (nch % NBUF), nch):
            step(c_tail, c_tail % NBUF)

        wait_scatter(nch - 2, (nch - 2) % 2)
        wait_scatter(nch - 1, (nch - 1) % 2)
        plsc.subcore_barrier()

        pltpu.sync_copy(acc.at[pl.ds(r0, RPT)],
                        out.at[cid].at[pl.ds(r0, RPT)])

    return agg


_sc_agg_halves = _sc_agg(False)
_sc_agg_partial = _sc_agg(True)

_R = 400


def _tc_call(body, out_shapes, out_specs, in_specs, *args):
    if len(out_shapes) == 1:
        out_shapes, out_specs = out_shapes[0], out_specs[0]
    return pl.pallas_call(
        body,
        grid=(N // _R,),
        in_specs=in_specs,
        out_specs=out_specs,
        out_shape=out_shapes,
    )(*args)


def _split_store(o_ref, y):
    o_ref[0] = y[:, :HD].astype(jnp.bfloat16)
    o_ref[1] = y[:, HD:].astype(jnp.bfloat16)


def _combine(p_ref, b_ref):
    return jnp.concatenate([p_ref[0], p_ref[1]], axis=1) + b_ref[...]


def _mm0_body(x_ref, w_ref, o_ref):
    _split_store(o_ref, jnp.dot(x_ref[...], w_ref[...],
                                preferred_element_type=jnp.float32))


def _mm_relu_keep_body(p_ref, b_ref, w_ref, h_ref, o_ref):
    h = jnp.maximum(_combine(p_ref, b_ref), 0.0)
    h_ref[...] = h
    _split_store(o_ref, jnp.dot(h, w_ref[...],
                                preferred_element_type=jnp.float32))


def _mm_relu_body(p_ref, b_ref, w_ref, o_ref):
    h = jnp.maximum(_combine(p_ref, b_ref), 0.0)
    _split_store(o_ref, jnp.dot(h, w_ref[...],
                                preferred_element_type=jnp.float32))


def _mm_relu_res_body(p_ref, b_ref, r_ref, w_ref, o_ref):
    h = jnp.maximum(_combine(p_ref, b_ref), 0.0) + r_ref[...]
    o_ref[0] = jnp.dot(h, w_ref[...],
                       preferred_element_type=jnp.float32).astype(jnp.bfloat16)


def _lsm_body(p_ref, b_ref, o_ref):
    z = (p_ref[0] + p_ref[1] + b_ref[...])[:, :40]
    m = jnp.max(z, axis=1, keepdims=True)
    ez = jnp.exp(z - m)
    lse = jnp.log(jnp.sum(ez, axis=1, keepdims=True)) + m
    o_ref[...] = z - lse


_spec_parts = pl.BlockSpec((NC, _R, HD), lambda i: (0, i, 0))


def _spec_full(r, c):
    return pl.BlockSpec((r, c), lambda i: (0, 0))


_row_spec = pl.BlockSpec((_R, 128), lambda i: (i, 0))
_split_spec = pl.BlockSpec((2, _R, HD), lambda i: (0, i, 0))
_split_shape = jax.ShapeDtypeStruct((2, N, HD), jnp.bfloat16)


def _interleave_perm(n):
    perm = np.empty((n,), np.int32)
    for blk in range(0, n, 2 * LANES):
        for k in range(LANES):
            perm[blk + 2 * k] = blk + k
            perm[blk + 2 * k + 1] = blk + LANES + k
    return perm


_PERM128 = _interleave_perm(128)
_PERM64 = _interleave_perm(HD)


def kernel(x, src, tgt, Mtgt, W0, b0, W1, b1, W2, b2, W3, b3):
    src_h = src.reshape(NS, E // NS // K, K)
    tgt_h = tgt.reshape(NS, E // NS // K, K)
    m_h = Mtgt.reshape(NS, E // NS // K, K)
    src_p = src.reshape(NW, E // NW // K, K)
    tgt_p = tgt.reshape(NW, E // NW // K, K)
    m_p = Mtgt.reshape(NW, E // NW // K, K)
    W0 = W0[:, _PERM128]
    W1 = W1[:, _PERM128]
    W2 = W2[:, _PERM128]
    W3p = jnp.pad(W3, ((0, 0), (0, HD - 40)))[:, _PERM64]
    b3p = jnp.pad(b3, (0, HD - 40))

    s0 = _tc_call(_mm0_body, [_split_shape], [_split_spec],
                  [_row_spec, _spec_full(128, 128)], x, W0)
    p0 = _sc_agg_halves(s0, src_h, tgt_h, m_h)

    h0, s1 = _tc_call(
        _mm_relu_keep_body,
        [jax.ShapeDtypeStruct((N, 128), jnp.float32), _split_shape],
        [_row_spec, _split_spec],
        [_spec_parts, _spec_full(1, 128), _spec_full(128, 128)],
        p0, b0.reshape(1, 128), W1)
    p1 = _sc_agg_halves(s1, src_h, tgt_h, m_h)

    s2 = _tc_call(_mm_relu_body, [_split_shape], [_split_spec],
                  [_spec_parts, _spec_full(1, 128), _spec_full(128, 128)],
                  p1, b1.reshape(1, 128), W2)
    p2 = _sc_agg_halves(s2, src_h, tgt_h, m_h)

    s3 = _tc_call(
        _mm_relu_res_body,
        [jax.ShapeDtypeStruct((1, N, HD), jnp.bfloat16)],
        [pl.BlockSpec((1, _R, HD), lambda i: (0, i, 0))],
        [_spec_parts, _spec_full(1, 128), _row_spec, _spec_full(128, HD)],
        p2, b2.reshape(1, 128), h0, W3p)
    p3 = _sc_agg_partial(s3, src_p, tgt_p, m_p)

    out = _tc_call(
        _lsm_body,
        [jax.ShapeDtypeStruct((N, 40), jnp.float32)],
        [pl.BlockSpec((_R, 40), lambda i: (i, 0))],
        [_spec_parts, _spec_full(1, HD)],
        p3, b3p.reshape(1, HD))
    return out


# speedup vs baseline: 1.1393x; 1.1393x over previous
import functools

import jax
import jax.numpy as jnp
import numpy as np
from jax import lax
from jax.experimental import pallas as pl
from jax.experimental.pallas import tpu as pltpu
from jax.experimental.pallas import tpu_sc as plsc

N = 10000
E = 320000
NC, NS, LANES = 2, 16, 16
NW = NC * NS
K = 80
NPAD = 10240
RPT = NPAD // NS
HD = 64
NBUF = 4


def _sc_agg(split_edges_across_cores):
    if split_edges_across_cores:
        nch = E // NW // K
        nh_sup = 1
    else:
        nch = E // NS // K
        nh_sup = 2
    mesh = plsc.VectorSubcoreMesh(core_axis_name="c", subcore_axis_name="s")

    @functools.partial(
        pl.kernel,
        out_type=jax.ShapeDtypeStruct((NC, NPAD, HD), jnp.bfloat16),
        mesh=mesh,
        compiler_params=pltpu.CompilerParams(use_tc_tiling_on_sc=False,
                                             needs_layout_passes=False),
        scratch_types=[
            pltpu.VMEM((nch, K), jnp.int32),
            pltpu.VMEM((nch, K), jnp.int32),
            pltpu.VMEM((nch, K), jnp.float32),
            pltpu.VMEM((NBUF, K, HD), jnp.bfloat16),
            pltpu.VMEM((2, K, HD), jnp.bfloat16),
            pltpu.VMEM_SHARED((NPAD, HD), jnp.bfloat16),
            pltpu.SemaphoreType.DMA,
            [pltpu.SemaphoreType.DMA for _ in range(NBUF)],
            [pltpu.SemaphoreType.DMA for _ in range(NBUF)],
        ],
    )
    def agg(sup, srcr, tgtr, mr, out, srcv, tgtv, mv, rows, frows, acc,
            sem_i, gsems, ssems):
        cid = lax.axis_index("c")
        sid = lax.axis_index("s")
        r0 = sid * RPT
        if split_edges_across_cores:
            erow = sid * NC + cid
            suph = sup.at[0]
        else:
            erow = sid
            suph = sup.at[cid]

        cp_s = pltpu.async_copy(srcr.at[erow], srcv, sem_i)
        cp_t = pltpu.async_copy(tgtr.at[erow], tgtv, sem_i)
        cp_m = pltpu.async_copy(mr.at[erow], mv, sem_i)

        zv = jnp.zeros((2 * LANES,), jnp.bfloat16)

        @pl.loop(0, K)
        def _(e):
            for j in range(HD // (2 * LANES)):
                frows[0, e, pl.ds(j * 2 * LANES, 2 * LANES)] = zv

        for j in range(RPT // K):
            pltpu.sync_copy(frows.at[0], acc.at[pl.ds(r0 + j * K, K)])

        cp_s.wait()
        cp_t.wait()
        cp_m.wait()
        plsc.subcore_barrier()

        def start_gather(c, b):
            pltpu.async_copy(suph.at[srcv.at[c]], rows.at[b], gsems[b])

        def wait_gather(c, b):
            pltpu.make_async_copy(suph.at[srcv.at[c]], rows.at[b],
                                  gsems[b]).wait()

        def start_scatter(c, bf):
            pltpu.async_copy(frows.at[bf], acc.at[tgtv.at[c]], ssems[bf],
                             add=True)

        def wait_scatter(c, bf):
            pltpu.make_async_copy(frows.at[bf], acc.at[tgtv.at[c]],
                                  ssems[bf]).wait()

        def scale(c, b, bf):
            @plsc.parallel_loop(0, K // LANES, unroll=2)
            def _(g):
                mvec = mv[c, pl.ds(g * LANES, LANES)]
                for i in range(LANES):
                    e = g * LANES + i
                    mf = jnp.full((LANES,), mvec[i])
                    m = plsc.pack(mf, mf, format=plsc.PackFormat.INTERLEAVED)
                    for j in range(HD // (2 * LANES)):
                        sl = pl.ds(j * 2 * LANES, 2 * LANES)
                        frows[bf, e, sl] = rows[b, e, sl] * m

        def step(c, b):
            bf = b % 2
            wait_gather(c, b)

            @pl.when(c >= 2)
            def _():
                wait_scatter(c - 2, bf)

            scale(c, b, bf)
            start_scatter(c, bf)

            @pl.when(c + 2 < nch)
            def _():
                start_gather(c + 2, (b + 2) % NBUF)

        start_gather(0, 0)
        start_gather(1, 1)

        @pl.loop(0, nch - (nch % NBUF), step=NBUF)
        def _(c):
            for db in range(NBUF):
                step(c + db, db)

        for c_tail in range(nch - (nch % NBUF), nch):
            step(c_tail, c_tail % NBUF)

        wait_scatter(nch - 2, (nch - 2) % 2)
        wait_scatter(nch - 1, (nch - 1) % 2)
        plsc.subcore_barrier()

        pltpu.sync_copy(acc.at[pl.ds(r0, RPT)],
                        out.at[cid].at[pl.ds(r0, RPT)])

    return agg


_sc_agg_halves = _sc_agg(False)
_sc_agg_partial = _sc_agg(True)

_R = 400


def _tc_call(body, out_shapes, out_specs, in_specs, *args):
    if len(out_shapes) == 1:
        out_shapes, out_specs = out_shapes[0], out_specs[0]
    return pl.pallas_call(
        body,
        grid=(N // _R,),
        in_specs=in_specs,
        out_specs=out_specs,
        out_shape=out_shapes,
    )(*args)


def _split_store(o_ref, y):
    o_ref[0] = y[:, :HD].astype(jnp.bfloat16)
    o_ref[1] = y[:, HD:].astype(jnp.bfloat16)


def _combine(p_ref, b_ref):
    return jnp.concatenate(
        [p_ref[0], p_ref[1]], axis=1).astype(jnp.float32) + b_ref[...]


def _mm0_body(x_ref, w_ref, o_ref):
    _split_store(o_ref, jnp.dot(x_ref[...], w_ref[...],
                                preferred_element_type=jnp.float32))


def _mm_relu_keep_body(p_ref, b_ref, w_ref, h_ref, o_ref):
    h = jnp.maximum(_combine(p_ref, b_ref), 0.0)
    h_ref[...] = h
    _split_store(o_ref, jnp.dot(h, w_ref[...],
                                preferred_element_type=jnp.float32))


def _mm_relu_body(p_ref, b_ref, w_ref, o_ref):
    h = jnp.maximum(_combine(p_ref, b_ref), 0.0)
    _split_store(o_ref, jnp.dot(h, w_ref[...],
                                preferred_element_type=jnp.float32))


def _mm_relu_res_body(p_ref, b_ref, r_ref, w_ref, o_ref):
    h = jnp.maximum(_combine(p_ref, b_ref), 0.0) + r_ref[...]
    o_ref[0] = jnp.dot(h, w_ref[...],
                       preferred_element_type=jnp.float32).astype(jnp.bfloat16)


def _lsm_body(p_ref, b_ref, o_ref):
    z = (p_ref[0].astype(jnp.float32) + p_ref[1].astype(jnp.float32)
         + b_ref[...])[:, :40]
    m = jnp.max(z, axis=1, keepdims=True)
    ez = jnp.exp(z - m)
    lse = jnp.log(jnp.sum(ez, axis=1, keepdims=True)) + m
    o_ref[...] = z - lse


_spec_parts = pl.BlockSpec((NC, _R, HD), lambda i: (0, i, 0))


def _spec_full(r, c):
    return pl.BlockSpec((r, c), lambda i: (0, 0))


_row_spec = pl.BlockSpec((_R, 128), lambda i: (i, 0))
_split_spec = pl.BlockSpec((2, _R, HD), lambda i: (0, i, 0))
_split_shape = jax.ShapeDtypeStruct((2, N, HD), jnp.bfloat16)


def _interleave_perm(n):
    perm = np.empty((n,), np.int32)
    for blk in range(0, n, 2 * LANES):
        for k in range(LANES):
            perm[blk + 2 * k] = blk + k
            perm[blk + 2 * k + 1] = blk + LANES + k
    return perm


_PERM128 = _interleave_perm(128)
_PERM64 = _interleave_perm(HD)


def kernel(x, src, tgt, Mtgt, W0, b0, W1, b1, W2, b2, W3, b3):
    src_h = src.reshape(NS, E // NS // K, K)
    tgt_h = tgt.reshape(NS, E // NS // K, K)
    m_h = Mtgt.reshape(NS, E // NS // K, K)
    src_p = src.reshape(NW, E // NW // K, K)
    tgt_p = tgt.reshape(NW, E // NW // K, K)
    m_p = Mtgt.reshape(NW, E // NW // K, K)
    W3p = jnp.pad(W3, ((0, 0), (0, HD - 40)))
    b3p = jnp.pad(b3, (0, HD - 40))

    s0 = _tc_call(_mm0_body, [_split_shape], [_split_spec],
                  [_row_spec, _spec_full(128, 128)], x, W0)
    p0 = _sc_agg_halves(s0, src_h, tgt_h, m_h)

    h0, s1 = _tc_call(
        _mm_relu_keep_body,
        [jax.ShapeDtypeStruct((N, 128), jnp.float32), _split_shape],
        [_row_spec, _split_spec],
        [_spec_parts, _spec_full(1, 128), _spec_full(128, 128)],
        p0, b0.reshape(1, 128), W1)
    p1 = _sc_agg_halves(s1, src_h, tgt_h, m_h)

    s2 = _tc_call(_mm_relu_body, [_split_shape], [_split_spec],
                  [_spec_parts, _spec_full(1, 128), _spec_full(128, 128)],
                  p1, b1.reshape(1, 128), W2)
    p2 = _sc_agg_halves(s2, src_h, tgt_h, m_h)

    s3 = _tc_call(
        _mm_relu_res_body,
        [jax.ShapeDtypeStruct((1, N, HD), jnp.bfloat16)],
        [pl.BlockSpec((1, _R, HD), lambda i: (0, i, 0))],
        [_spec_parts, _spec_full(1, 128), _row_spec, _spec_full(128, HD)],
        p2, b2.reshape(1, 128), h0, W3p)
    p3 = _sc_agg_partial(s3, src_p, tgt_p, m_p)

    out = _tc_call(
        _lsm_body,
        [jax.ShapeDtypeStruct((N, 40), jnp.float32)],
        [pl.BlockSpec((_R, 40), lambda i: (i, 0))],
        [_spec_parts, _spec_full(1, HD)],
        p3, b3p.reshape(1, HD))
    return out


# speedup vs baseline: 1.3556x; 1.1899x over previous
import functools

import jax
import jax.numpy as jnp
import numpy as np
from jax import lax
from jax.experimental import pallas as pl
from jax.experimental.pallas import tpu as pltpu
from jax.experimental.pallas import tpu_sc as plsc

N = 10000
E = 320000
NC, NS, LANES = 2, 16, 16
NW = NC * NS
K = 80
NPAD = 10240
RPT = NPAD // NS
HD = 64
NBUF = 4


def _sc_agg(split_edges_across_cores):
    if split_edges_across_cores:
        nch = E // NW // K
        nh_sup = 1
    else:
        nch = E // NS // K
        nh_sup = 2
    mesh = plsc.VectorSubcoreMesh(core_axis_name="c", subcore_axis_name="s")

    @functools.partial(
        pl.kernel,
        out_type=jax.ShapeDtypeStruct((NC, NPAD, HD), jnp.bfloat16),
        mesh=mesh,
        compiler_params=pltpu.CompilerParams(use_tc_tiling_on_sc=False,
                                             needs_layout_passes=False),
        scratch_types=[
            pltpu.VMEM((nch, K), jnp.int32),
            pltpu.VMEM((nch, K), jnp.int32),
            pltpu.VMEM((nch, K), jnp.float32),
            pltpu.VMEM((NBUF, K, HD), jnp.bfloat16),
            pltpu.VMEM((2, K, HD), jnp.bfloat16),
            pltpu.VMEM_SHARED((NPAD, HD), jnp.bfloat16),
            pltpu.SemaphoreType.DMA,
            [pltpu.SemaphoreType.DMA for _ in range(NBUF)],
            [pltpu.SemaphoreType.DMA for _ in range(NBUF)],
        ],
    )
    def agg(sup, srcr, tgtr, mr, out, srcv, tgtv, mv, rows, frows, acc,
            sem_i, gsems, ssems):
        cid = lax.axis_index("c")
        sid = lax.axis_index("s")
        r0 = sid * RPT
        if split_edges_across_cores:
            erow = sid * NC + cid
            suph = sup.at[0]
        else:
            erow = sid
            suph = sup.at[cid]

        cp_s = pltpu.async_copy(srcr.at[erow], srcv, sem_i)
        cp_t = pltpu.async_copy(tgtr.at[erow], tgtv, sem_i)
        cp_m = pltpu.async_copy(mr.at[erow], mv, sem_i)

        zv = jnp.zeros((2 * LANES,), jnp.bfloat16)

        @pl.loop(0, K)
        def _(e):
            for j in range(HD // (2 * LANES)):
                frows[0, e, pl.ds(j * 2 * LANES, 2 * LANES)] = zv

        for j in range(RPT // K):
            pltpu.sync_copy(frows.at[0], acc.at[pl.ds(r0 + j * K, K)])

        cp_s.wait()
        cp_t.wait()
        cp_m.wait()
        plsc.subcore_barrier()

        def start_gather(c, b):
            pltpu.async_copy(suph.at[srcv.at[c]], rows.at[b], gsems[b])

        def wait_gather(c, b):
            pltpu.make_async_copy(suph.at[srcv.at[c]], rows.at[b],
                                  gsems[b]).wait()

        def start_scatter(c, bf):
            pltpu.async_copy(frows.at[bf], acc.at[tgtv.at[c]], ssems[bf],
                             add=True)

        def wait_scatter(c, bf):
            pltpu.make_async_copy(frows.at[bf], acc.at[tgtv.at[c]],
                                  ssems[bf]).wait()

        def scale(c, b, bf):
            @plsc.parallel_loop(0, K // LANES, unroll=2)
            def _(g):
                mvec = mv[c, pl.ds(g * LANES, LANES)]
                for i in range(LANES):
                    e = g * LANES + i
                    mf = jnp.full((LANES,), mvec[i])
                    m = plsc.pack(mf, mf, format=plsc.PackFormat.INTERLEAVED)
                    for j in range(HD // (2 * LANES)):
                        sl = pl.ds(j * 2 * LANES, 2 * LANES)
                        frows[bf, e, sl] = rows[b, e, sl] * m

        def step(c, b):
            bf = b % 2
            wait_gather(c, b)

            @pl.when(c >= 2)
            def _():
                wait_scatter(c - 2, bf)

            scale(c, b, bf)
            start_scatter(c, bf)

            @pl.when(c + 3 < nch)
            def _():
                start_gather(c + 3, (b + 3) % NBUF)

        start_gather(0, 0)
        start_gather(1, 1)
        start_gather(2, 2)

        @pl.loop(0, nch - (nch % NBUF), step=NBUF)
        def _(c):
            for db in range(NBUF):
                step(c + db, db)

        for c_tail in range(nch - (nch % NBUF), nch):
            step(c_tail, c_tail % NBUF)

        wait_scatter(nch - 2, (nch - 2) % 2)
        wait_scatter(nch - 1, (nch - 1) % 2)
        plsc.subcore_barrier()

        pltpu.sync_copy(acc.at[pl.ds(r0, RPT)],
                        out.at[cid].at[pl.ds(r0, RPT)])

    return agg


_sc_agg_halves = _sc_agg(False)
_sc_agg_partial = _sc_agg(True)

_R = 400


def _tc_call(body, out_shapes, out_specs, in_specs, *args):
    if len(out_shapes) == 1:
        out_shapes, out_specs = out_shapes[0], out_specs[0]
    return pl.pallas_call(
        body,
        grid=(N // _R,),
        in_specs=in_specs,
        out_specs=out_specs,
        out_shape=out_shapes,
    )(*args)


def _split_store(o_ref, y):
    o_ref[0] = y[:, :HD].astype(jnp.bfloat16)
    o_ref[1] = y[:, HD:].astype(jnp.bfloat16)


def _combine(p_ref, b_ref):
    return jnp.concatenate(
        [p_ref[0], p_ref[1]], axis=1).astype(jnp.float32) + b_ref[...]


def _mm0_body(x_ref, w_ref, o_ref):
    _split_store(o_ref, jnp.dot(x_ref[...], w_ref[...],
                                preferred_element_type=jnp.float32))


def _mm_relu_keep_body(p_ref, b_ref, w_ref, h_ref, o_ref):
    h = jnp.maximum(_combine(p_ref, b_ref), 0.0)
    h_ref[...] = h
    _split_store(o_ref, jnp.dot(h, w_ref[...],
                                preferred_element_type=jnp.float32))


def _mm_relu_body(p_ref, b_ref, w_ref, o_ref):
    h = jnp.maximum(_combine(p_ref, b_ref), 0.0)
    _split_store(o_ref, jnp.dot(h, w_ref[...],
                                preferred_element_type=jnp.float32))


def _mm_relu_res_body(p_ref, b_ref, r_ref, w_ref, o_ref):
    h = jnp.maximum(_combine(p_ref, b_ref), 0.0) + r_ref[...]
    o_ref[0] = jnp.dot(h, w_ref[...],
                       preferred_element_type=jnp.float32).astype(jnp.bfloat16)


def _lsm_body(p_ref, b_ref, o_ref):
    z = (p_ref[0].astype(jnp.float32) + p_ref[1].astype(jnp.float32)
         + b_ref[...])[:, :40]
    m = jnp.max(z, axis=1, keepdims=True)
    ez = jnp.exp(z - m)
    lse = jnp.log(jnp.sum(ez, axis=1, keepdims=True)) + m
    o_ref[...] = z - lse


_spec_parts = pl.BlockSpec((NC, _R, HD), lambda i: (0, i, 0))


def _spec_full(r, c):
    return pl.BlockSpec((r, c), lambda i: (0, 0))


_row_spec = pl.BlockSpec((_R, 128), lambda i: (i, 0))
_split_spec = pl.BlockSpec((2, _R, HD), lambda i: (0, i, 0))
_split_shape = jax.ShapeDtypeStruct((2, N, HD), jnp.bfloat16)


def _interleave_perm(n):
    perm = np.empty((n,), np.int32)
    for blk in range(0, n, 2 * LANES):
        for k in range(LANES):
            perm[blk + 2 * k] = blk + k
            perm[blk + 2 * k + 1] = blk + LANES + k
    return perm


_PERM128 = _interleave_perm(128)
_PERM64 = _interleave_perm(HD)


def kernel(x, src, tgt, Mtgt, W0, b0, W1, b1, W2, b2, W3, b3):
    src_h = src.reshape(NS, E // NS // K, K)
    tgt_h = tgt.reshape(NS, E // NS // K, K)
    m_h = Mtgt.reshape(NS, E // NS // K, K)
    src_p = src.reshape(NW, E // NW // K, K)
    tgt_p = tgt.reshape(NW, E // NW // K, K)
    m_p = Mtgt.reshape(NW, E // NW // K, K)
    W3p = jnp.pad(W3, ((0, 0), (0, HD - 40)))
    b3p = jnp.pad(b3, (0, HD - 40))

    s0 = _tc_call(_mm0_body, [_split_shape], [_split_spec],
                  [_row_spec, _spec_full(128, 128)], x, W0)
    p0 = _sc_agg_halves(s0, src_h, tgt_h, m_h)

    h0, s1 = _tc_call(
        _mm_relu_keep_body,
        [jax.ShapeDtypeStruct((N, 128), jnp.float32), _split_shape],
        [_row_spec, _split_spec],
        [_spec_parts, _spec_full(1, 128), _spec_full(128, 128)],
        p0, b0.reshape(1, 128), W1)
    p1 = _sc_agg_halves(s1, src_h, tgt_h, m_h)

    s2 = _tc_call(_mm_relu_body, [_split_shape], [_split_spec],
                  [_spec_parts, _spec_full(1, 128), _spec_full(128, 128)],
                  p1, b1.reshape(1, 128), W2)
    p2 = _sc_agg_halves(s2, src_h, tgt_h, m_h)

    s3 = _tc_call(
        _mm_relu_res_body,
        [jax.ShapeDtypeStruct((1, N, HD), jnp.bfloat16)],
        [pl.BlockSpec((1, _R, HD), lambda i: (0, i, 0))],
        [_spec_parts, _spec_full(1, 128), _row_spec, _spec_full(128, HD)],
        p2, b2.reshape(1, 128), h0, W3p)
    p3 = _sc_agg_partial(s3, src_p, tgt_p, m_p)

    out = _tc_call(
        _lsm_body,
        [jax.ShapeDtypeStruct((N, 40), jnp.float32)],
        [pl.BlockSpec((_R, 40), lambda i: (i, 0))],
        [_spec_parts, _spec_full(1, HD)],
        p3, b3p.reshape(1, HD))
    return out


# speedup vs baseline: 1.6170x; 1.1928x over previous
import functools

import jax
import jax.numpy as jnp
import numpy as np
from jax import lax
from jax.experimental import pallas as pl
from jax.experimental.pallas import tpu as pltpu
from jax.experimental.pallas import tpu_sc as plsc

N = 10000
E = 320000
NC, NS, LANES = 2, 16, 16
NW = NC * NS
K = 80
NPAD = 10240
RPT = NPAD // NS
HD = 64
NBUF = 6


def _sc_agg(split_edges_across_cores):
    if split_edges_across_cores:
        nch = E // NW // K
        nh_sup = 1
    else:
        nch = E // NS // K
        nh_sup = 2
    mesh = plsc.VectorSubcoreMesh(core_axis_name="c", subcore_axis_name="s")

    @functools.partial(
        pl.kernel,
        out_type=jax.ShapeDtypeStruct((NC, NPAD, HD), jnp.bfloat16),
        mesh=mesh,
        compiler_params=pltpu.CompilerParams(use_tc_tiling_on_sc=False,
                                             needs_layout_passes=False),
        scratch_types=[
            pltpu.VMEM((nch, K), jnp.int32),
            pltpu.VMEM((nch, K), jnp.int32),
            pltpu.VMEM((nch, K), jnp.float32),
            pltpu.VMEM((NBUF, K, HD), jnp.bfloat16),
            pltpu.VMEM((2, K, HD), jnp.bfloat16),
            pltpu.VMEM_SHARED((NPAD, HD), jnp.bfloat16),
            pltpu.SemaphoreType.DMA,
            [pltpu.SemaphoreType.DMA for _ in range(NBUF)],
            [pltpu.SemaphoreType.DMA for _ in range(NBUF)],
        ],
    )
    def agg(sup, srcr, tgtr, mr, out, srcv, tgtv, mv, rows, frows, acc,
            sem_i, gsems, ssems):
        cid = lax.axis_index("c")
        sid = lax.axis_index("s")
        r0 = sid * RPT
        if split_edges_across_cores:
            erow = sid * NC + cid
            suph = sup.at[0]
        else:
            erow = sid
            suph = sup.at[cid]

        cp_s = pltpu.async_copy(srcr.at[erow], srcv, sem_i)
        cp_t = pltpu.async_copy(tgtr.at[erow], tgtv, sem_i)
        cp_m = pltpu.async_copy(mr.at[erow], mv, sem_i)

        zv = jnp.zeros((2 * LANES,), jnp.bfloat16)

        @pl.loop(0, K)
        def _(e):
            for j in range(HD // (2 * LANES)):
                frows[0, e, pl.ds(j * 2 * LANES, 2 * LANES)] = zv

        for j in range(RPT // K):
            pltpu.sync_copy(frows.at[0], acc.at[pl.ds(r0 + j * K, K)])

        cp_s.wait()
        cp_t.wait()
        cp_m.wait()
        plsc.subcore_barrier()

        def start_gather(c, b):
            pltpu.async_copy(suph.at[srcv.at[c]], rows.at[b], gsems[b])

        def wait_gather(c, b):
            pltpu.make_async_copy(suph.at[srcv.at[c]], rows.at[b],
                                  gsems[b]).wait()

        def start_scatter(c, bf):
            pltpu.async_copy(frows.at[bf], acc.at[tgtv.at[c]], ssems[bf],
                             add=True)

        def wait_scatter(c, bf):
            pltpu.make_async_copy(frows.at[bf], acc.at[tgtv.at[c]],
                                  ssems[bf]).wait()

        def scale(c, b, bf):
            @plsc.parallel_loop(0, K // LANES, unroll=2)
            def _(g):
                mvec = mv[c, pl.ds(g * LANES, LANES)]
                for i in range(LANES):
                    e = g * LANES + i
                    mf = jnp.full((LANES,), mvec[i])
                    m = plsc.pack(mf, mf, format=plsc.PackFormat.INTERLEAVED)
                    for j in range(HD // (2 * LANES)):
                        sl = pl.ds(j * 2 * LANES, 2 * LANES)
                        frows[bf, e, sl] = rows[b, e, sl] * m

        def step(c, b):
            bf = b % 2
            wait_gather(c, b)

            @pl.when(c >= 2)
            def _():
                wait_scatter(c - 2, bf)

            scale(c, b, bf)
            start_scatter(c, bf)

            @pl.when(c + 5 < nch)
            def _():
                start_gather(c + 5, (b + 5) % NBUF)

        for cp_ in range(5):
            start_gather(cp_, cp_)

        @pl.loop(0, nch - (nch % NBUF), step=NBUF)
        def _(c):
            for db in range(NBUF):
                step(c + db, db)

        for c_tail in range(nch - (nch % NBUF), nch):
            step(c_tail, c_tail % NBUF)

        wait_scatter(nch - 2, (nch - 2) % 2)
        wait_scatter(nch - 1, (nch - 1) % 2)
        plsc.subcore_barrier()

        pltpu.sync_copy(acc.at[pl.ds(r0, RPT)],
                        out.at[cid].at[pl.ds(r0, RPT)])

    return agg


_sc_agg_halves = _sc_agg(False)
_sc_agg_partial = _sc_agg(True)

_R = 400


def _tc_call(body, out_shapes, out_specs, in_specs, *args):
    if len(out_shapes) == 1:
        out_shapes, out_specs = out_shapes[0], out_specs[0]
    return pl.pallas_call(
        body,
        grid=(N // _R,),
        in_specs=in_specs,
        out_specs=out_specs,
        out_shape=out_shapes,
    )(*args)


def _split_store(o_ref, y):
    o_ref[0] = y[:, :HD].astype(jnp.bfloat16)
    o_ref[1] = y[:, HD:].astype(jnp.bfloat16)


def _combine(p_ref, b_ref):
    return jnp.concatenate(
        [p_ref[0], p_ref[1]], axis=1).astype(jnp.float32) + b_ref[...]


def _mm0_body(x_ref, w_ref, o_ref):
    _split_store(o_ref, jnp.dot(x_ref[...], w_ref[...],
                                preferred_element_type=jnp.float32))


def _mm_relu_keep_body(p_ref, b_ref, w_ref, h_ref, o_ref):
    h = jnp.maximum(_combine(p_ref, b_ref), 0.0)
    h_ref[...] = h
    _split_store(o_ref, jnp.dot(h, w_ref[...],
                                preferred_element_type=jnp.float32))


def _mm_relu_body(p_ref, b_ref, w_ref, o_ref):
    h = jnp.maximum(_combine(p_ref, b_ref), 0.0)
    _split_store(o_ref, jnp.dot(h, w_ref[...],
                                preferred_element_type=jnp.float32))


def _mm_relu_res_body(p_ref, b_ref, r_ref, w_ref, o_ref):
    h = jnp.maximum(_combine(p_ref, b_ref), 0.0) + r_ref[...]
    o_ref[0] = jnp.dot(h, w_ref[...],
                       preferred_element_type=jnp.float32).astype(jnp.bfloat16)


def _lsm_body(p_ref, b_ref, o_ref):
    z = (p_ref[0].astype(jnp.float32) + p_ref[1].astype(jnp.float32)
         + b_ref[...])[:, :40]
    m = jnp.max(z, axis=1, keepdims=True)
    ez = jnp.exp(z - m)
    lse = jnp.log(jnp.sum(ez, axis=1, keepdims=True)) + m
    o_ref[...] = z - lse


_spec_parts = pl.BlockSpec((NC, _R, HD), lambda i: (0, i, 0))


def _spec_full(r, c):
    return pl.BlockSpec((r, c), lambda i: (0, 0))


_row_spec = pl.BlockSpec((_R, 128), lambda i: (i, 0))
_split_spec = pl.BlockSpec((2, _R, HD), lambda i: (0, i, 0))
_split_shape = jax.ShapeDtypeStruct((2, N, HD), jnp.bfloat16)


def _interleave_perm(n):
    perm = np.empty((n,), np.int32)
    for blk in range(0, n, 2 * LANES):
        for k in range(LANES):
            perm[blk + 2 * k] = blk + k
            perm[blk + 2 * k + 1] = blk + LANES + k
    return perm


_PERM128 = _interleave_perm(128)
_PERM64 = _interleave_perm(HD)


def kernel(x, src, tgt, Mtgt, W0, b0, W1, b1, W2, b2, W3, b3):
    src_h = src.reshape(NS, E // NS // K, K)
    tgt_h = tgt.reshape(NS, E // NS // K, K)
    m_h = Mtgt.reshape(NS, E // NS // K, K)
    src_p = src.reshape(NW, E // NW // K, K)
    tgt_p = tgt.reshape(NW, E // NW // K, K)
    m_p = Mtgt.reshape(NW, E // NW // K, K)
    W3p = jnp.pad(W3, ((0, 0), (0, HD - 40)))
    b3p = jnp.pad(b3, (0, HD - 40))

    s0 = _tc_call(_mm0_body, [_split_shape], [_split_spec],
                  [_row_spec, _spec_full(128, 128)], x, W0)
    p0 = _sc_agg_halves(s0, src_h, tgt_h, m_h)

    h0, s1 = _tc_call(
        _mm_relu_keep_body,
        [jax.ShapeDtypeStruct((N, 128), jnp.float32), _split_shape],
        [_row_spec, _split_spec],
        [_spec_parts, _spec_full(1, 128), _spec_full(128, 128)],
        p0, b0.reshape(1, 128), W1)
    p1 = _sc_agg_halves(s1, src_h, tgt_h, m_h)

    s2 = _tc_call(_mm_relu_body, [_split_shape], [_split_spec],
                  [_spec_parts, _spec_full(1, 128), _spec_full(128, 128)],
                  p1, b1.reshape(1, 128), W2)
    p2 = _sc_agg_halves(s2, src_h, tgt_h, m_h)

    s3 = _tc_call(
        _mm_relu_res_body,
        [jax.ShapeDtypeStruct((1, N, HD), jnp.bfloat16)],
        [pl.BlockSpec((1, _R, HD), lambda i: (0, i, 0))],
        [_spec_parts, _spec_full(1, 128), _row_spec, _spec_full(128, HD)],
        p2, b2.reshape(1, 128), h0, W3p)
    p3 = _sc_agg_partial(s3, src_p, tgt_p, m_p)

    out = _tc_call(
        _lsm_body,
        [jax.ShapeDtypeStruct((N, 40), jnp.float32)],
        [pl.BlockSpec((_R, 40), lambda i: (i, 0))],
        [_spec_parts, _spec_full(1, HD)],
        p3, b3p.reshape(1, HD))
    return out
